# baseline (speedup 1.0000x reference)
import jax
import jax.numpy as jnp
from jax.experimental import pallas as pl
from jax.experimental.pallas import tpu as pltpu

N = 256
D = 256
E = 4096
EB = 512
NBLK = E // EB
EPS = 1e-06
F32 = jnp.float32


def _fused_body(sub_ref, obj_ref, rel_ref, sub_all_ref, obj_all_ref, phr_ref,
                wrel_ref, wsub_ref, wobj_ref, wphr_ref, brel_ref, bsub_ref,
                bobj_ref, bphr_ref, t_ref, upd_ref, out_ref, tsub_s, tobj_s,
                atte_s, upd_s, erow_s, ecol_s):
    i = pl.program_id(0)

    @pl.when(i == 0)
    def _tables():
        phr = phr_ref[...]
        tsub_s[:, :D] = jnp.dot(phr, wrel_ref[:D], preferred_element_type=F32)
        tsub_s[:, D:] = jnp.dot(phr, wsub_ref[:D], preferred_element_type=F32)
        tobj_s[:, :D] = jnp.dot(phr, wrel_ref[D:2 * D],
                                preferred_element_type=F32)
        tobj_s[:, D:] = jnp.dot(phr, wobj_ref[:D], preferred_element_type=F32)
        t = t_ref[0, 0]
        sub_all = sub_all_ref[0, :]
        obj_all = obj_all_ref[0, :]
        iota_e = jax.lax.broadcasted_iota(jnp.int32, (E, N), 0)
        iota_o = jax.lax.broadcasted_iota(jnp.int32, (E, N), 1)
        subc = sub_all[:, None]
        objc = obj_all[:, None]
        rowval = jnp.where(subc == t, iota_e[:, 0:1], -1)
        colval = jnp.where(objc == t, iota_e[:, 0:1], -1)
        erow_s[0, :] = jnp.max(jnp.where(objc == iota_o, rowval, -1), axis=0)
        ecol_s[0, :] = jnp.max(jnp.where(subc == iota_o, colval, -1), axis=0)

    sub = sub_ref[0, :]
    obj = obj_ref[0, :]
    iota_n = jax.lax.broadcasted_iota(jnp.int32, (EB, N), 1)
    oh_sub = (sub[:, None] == iota_n).astype(F32)
    oh_obj = (obj[:, None] == iota_n).astype(F32)
    gsub = jnp.dot(oh_sub, tsub_s[...], preferred_element_type=F32)
    gobj = jnp.dot(oh_obj, tobj_s[...], preferred_element_type=F32)
    upd = (gsub[:, :D] + gobj[:, :D] + brel_ref[...]
           + jnp.dot(rel_ref[...], wrel_ref[2 * D:],
                     preferred_element_type=F32))
    upd_ref[...] = upd
    upd_s[pl.ds(i * EB, EB), :] = upd
    ts = gsub[:, D:] + bsub_ref[...] + jnp.dot(upd, wsub_ref[D:],
                                               preferred_element_type=F32)
    to = gobj[:, D:] + bobj_ref[...] + jnp.dot(upd, wobj_ref[D:],
                                               preferred_element_type=F32)
    ones_col = jnp.ones((D, 1), dtype=F32)
    atte_s[pl.ds(i * EB, EB), :] = jnp.dot(ts * to, ones_col,
                                           preferred_element_type=F32) * (
                                               1.0 / (D ** 0.5))

    @pl.when(i == NBLK - 1)
    def _context():
        t = t_ref[0, 0]
        iota_e = jax.lax.broadcasted_iota(jnp.int32, (E, N), 0)
        e_row = erow_s[0, :]
        e_col = ecol_s[0, :]
        sel_row = (iota_e == e_row[None, :]).astype(F32)
        sel_col = (iota_e == e_col[None, :]).astype(F32)
        atte_col = atte_s[...]
        a_row = jax.lax.dot_general(atte_col, sel_row, (((0,), (0,)), ((), ())),
                                    preferred_element_type=F32)[0]
        a_col = jax.lax.dot_general(atte_col, sel_col, (((0,), (0,)), ((), ())),
                                    preferred_element_type=F32)[0]
        mask_row = (e_row >= 0).astype(F32)
        mask_col = (e_col >= 0).astype(F32)

        def msm(vec, mask):
            mv = vec * mask
            ex = jnp.exp(mv - jnp.max(mv)) * mask
            return ex / (jnp.sum(ex) + EPS)

        w_row = msm(a_row, mask_row)
        w_col = msm(a_col, mask_col)
        weff = (jnp.dot(sel_row, w_row[:, None], preferred_element_type=F32)
                + jnp.dot(sel_col, w_col[:, None], preferred_element_type=F32))
        ctx1 = jnp.dot((w_row + w_col)[None, :], phr_ref[...],
                       preferred_element_type=F32)
        ctx2 = jax.lax.dot_general(weff, upd_s[...], (((0,), (0,)), ((), ())),
                                   preferred_element_type=F32)
        delta = (jnp.dot(ctx1, wphr_ref[:D], preferred_element_type=F32)
                 + jnp.dot(ctx2, wphr_ref[D:], preferred_element_type=F32)
                 + bphr_ref[...])
        row_is_t = jax.lax.broadcasted_iota(jnp.int32, (N, 1), 0) == t
        out_ref[...] = phr_ref[...] + jnp.where(row_is_t, delta, 0.0)


@jax.jit
def _run(phrase_feat, rel_feat, rel_conn_mat, target_id, W_rel, b_rel, W_sub,
         b_sub, W_obj, b_obj, W_phr, b_phr):
    sub = rel_conn_mat[0:1].astype(jnp.int32)
    obj = rel_conn_mat[1:2].astype(jnp.int32)
    t = jnp.asarray(target_id, jnp.int32).reshape(1, 1)
    brel = b_rel.reshape(1, D)
    bsub = b_sub.reshape(1, D)
    bobj = b_obj.reshape(1, D)
    bphr = b_phr.reshape(1, D)

    full = lambda shape: pl.BlockSpec(shape, lambda i: tuple(0 for _ in shape))
    upd, out1 = pl.pallas_call(
        _fused_body,
        grid=(NBLK,),
        in_specs=[
            pl.BlockSpec((1, EB), lambda i: (0, i)),
            pl.BlockSpec((1, EB), lambda i: (0, i)),
            pl.BlockSpec((EB, D), lambda i: (i, 0)),
            full((1, E)),
            full((1, E)),
            full((N, D)),
            full((3 * D, D)),
            full((2 * D, D)),
            full((2 * D, D)),
            full((2 * D, D)),
            full((1, D)),
            full((1, D)),
            full((1, D)),
            full((1, D)),
            full((1, 1)),
        ],
        out_specs=(pl.BlockSpec((EB, D), lambda i: (i, 0)), full((N, D))),
        out_shape=(jax.ShapeDtypeStruct((E, D), F32),
                   jax.ShapeDtypeStruct((N, D), F32)),
        scratch_shapes=[
            pltpu.VMEM((N, 2 * D), F32),
            pltpu.VMEM((N, 2 * D), F32),
            pltpu.VMEM((E, 1), F32),
            pltpu.VMEM((E, D), F32),
            pltpu.VMEM((1, N), jnp.int32),
            pltpu.VMEM((1, N), jnp.int32),
        ],
    )(sub, obj, rel_feat, sub, obj, phrase_feat, W_rel, W_sub, W_obj, W_phr,
      brel, bsub, bobj, bphr, t)
    return out1, upd


def kernel(phrase_feat, rel_feat, rel_conn_mat, target_id, W_rel, b_rel,
           W_sub, b_sub, W_obj, b_obj, W_phr, b_phr):
    return _run(phrase_feat, rel_feat, rel_conn_mat, target_id, W_rel, b_rel,
                W_sub, b_sub, W_obj, b_obj, W_phr, b_phr)


# speedup vs baseline: 1.1365x; 1.1365x over previous
import jax
import jax.numpy as jnp
from jax.experimental import pallas as pl
from jax.experimental.pallas import tpu as pltpu

N = 256
D = 256
E = 4096
EB = 1024
NBLK = E // EB
EPS = 1e-06
F32 = jnp.float32


def _fused_body(sub_ref, obj_ref, rel_ref, sub_all_ref, obj_all_ref, phr_ref,
                wrel_ref, wsub_ref, wobj_ref, wphr_ref, brel_ref, bsub_ref,
                bobj_ref, bphr_ref, t_ref, upd_ref, out_ref, tsub_s, tobj_s,
                atte_s, upd_s, erow_s, ecol_s):
    i = pl.program_id(0)

    @pl.when(i == 0)
    def _tables():
        phr = phr_ref[...]
        tsub_s[:, :D] = jnp.dot(phr, wrel_ref[:D], preferred_element_type=F32)
        tsub_s[:, D:] = jnp.dot(phr, wsub_ref[:D], preferred_element_type=F32)
        tobj_s[:, :D] = jnp.dot(phr, wrel_ref[D:2 * D],
                                preferred_element_type=F32)
        tobj_s[:, D:] = jnp.dot(phr, wobj_ref[:D], preferred_element_type=F32)
        t = t_ref[0, 0]
        sub_all = sub_all_ref[0, :]
        obj_all = obj_all_ref[0, :]
        iota_e = jax.lax.broadcasted_iota(jnp.int32, (E, N), 0)
        iota_o = jax.lax.broadcasted_iota(jnp.int32, (E, N), 1)
        subc = sub_all[:, None]
        objc = obj_all[:, None]
        rowval = jnp.where(subc == t, iota_e[:, 0:1], -1)
        colval = jnp.where(objc == t, iota_e[:, 0:1], -1)
        erow_s[0, :] = jnp.max(jnp.where(objc == iota_o, rowval, -1), axis=0)
        ecol_s[0, :] = jnp.max(jnp.where(subc == iota_o, colval, -1), axis=0)

    sub = sub_ref[0, :]
    obj = obj_ref[0, :]
    iota_n = jax.lax.broadcasted_iota(jnp.int32, (EB, N), 1)
    oh_sub = (sub[:, None] == iota_n).astype(F32)
    oh_obj = (obj[:, None] == iota_n).astype(F32)
    gsub = jnp.dot(oh_sub, tsub_s[...], preferred_element_type=F32)
    gobj = jnp.dot(oh_obj, tobj_s[...], preferred_element_type=F32)
    upd = (gsub[:, :D] + gobj[:, :D] + brel_ref[...]
           + jnp.dot(rel_ref[...], wrel_ref[2 * D:],
                     preferred_element_type=F32))
    upd_ref[...] = upd
    upd_s[pl.ds(i * EB, EB), :] = upd
    ts = gsub[:, D:] + bsub_ref[...] + jnp.dot(upd, wsub_ref[D:],
                                               preferred_element_type=F32)
    to = gobj[:, D:] + bobj_ref[...] + jnp.dot(upd, wobj_ref[D:],
                                               preferred_element_type=F32)
    ones_col = jnp.ones((D, 1), dtype=F32)
    atte_s[pl.ds(i * EB, EB), :] = jnp.dot(ts * to, ones_col,
                                           preferred_element_type=F32) * (
                                               1.0 / (D ** 0.5))

    @pl.when(i == NBLK - 1)
    def _context():
        t = t_ref[0, 0]
        iota_e = jax.lax.broadcasted_iota(jnp.int32, (E, N), 0)
        e_row = erow_s[0, :]
        e_col = ecol_s[0, :]
        sel_row = (iota_e == e_row[None, :]).astype(F32)
        sel_col = (iota_e == e_col[None, :]).astype(F32)
        atte_col = atte_s[...]
        a_row = jax.lax.dot_general(atte_col, sel_row, (((0,), (0,)), ((), ())),
                                    preferred_element_type=F32)[0]
        a_col = jax.lax.dot_general(atte_col, sel_col, (((0,), (0,)), ((), ())),
                                    preferred_element_type=F32)[0]
        mask_row = (e_row >= 0).astype(F32)
        mask_col = (e_col >= 0).astype(F32)

        def msm(vec, mask):
            mv = vec * mask
            ex = jnp.exp(mv - jnp.max(mv)) * mask
            return ex / (jnp.sum(ex) + EPS)

        w_row = msm(a_row, mask_row)
        w_col = msm(a_col, mask_col)
        weff = (jnp.dot(sel_row, w_row[:, None], preferred_element_type=F32)
                + jnp.dot(sel_col, w_col[:, None], preferred_element_type=F32))
        ctx1 = jnp.dot((w_row + w_col)[None, :], phr_ref[...],
                       preferred_element_type=F32)
        ctx2 = jax.lax.dot_general(weff, upd_s[...], (((0,), (0,)), ((), ())),
                                   preferred_element_type=F32)
        delta = (jnp.dot(ctx1, wphr_ref[:D], preferred_element_type=F32)
                 + jnp.dot(ctx2, wphr_ref[D:], preferred_element_type=F32)
                 + bphr_ref[...])
        row_is_t = jax.lax.broadcasted_iota(jnp.int32, (N, 1), 0) == t
        out_ref[...] = phr_ref[...] + jnp.where(row_is_t, delta, 0.0)


@jax.jit
def _run(phrase_feat, rel_feat, rel_conn_mat, target_id, W_rel, b_rel, W_sub,
         b_sub, W_obj, b_obj, W_phr, b_phr):
    sub = rel_conn_mat[0:1].astype(jnp.int32)
    obj = rel_conn_mat[1:2].astype(jnp.int32)
    t = jnp.asarray(target_id, jnp.int32).reshape(1, 1)
    brel = b_rel.reshape(1, D)
    bsub = b_sub.reshape(1, D)
    bobj = b_obj.reshape(1, D)
    bphr = b_phr.reshape(1, D)

    full = lambda shape: pl.BlockSpec(shape, lambda i: tuple(0 for _ in shape))
    upd, out1 = pl.pallas_call(
        _fused_body,
        grid=(NBLK,),
        in_specs=[
            pl.BlockSpec((1, EB), lambda i: (0, i)),
            pl.BlockSpec((1, EB), lambda i: (0, i)),
            pl.BlockSpec((EB, D), lambda i: (i, 0)),
            full((1, E)),
            full((1, E)),
            full((N, D)),
            full((3 * D, D)),
            full((2 * D, D)),
            full((2 * D, D)),
            full((2 * D, D)),
            full((1, D)),
            full((1, D)),
            full((1, D)),
            full((1, D)),
            full((1, 1)),
        ],
        out_specs=(pl.BlockSpec((EB, D), lambda i: (i, 0)), full((N, D))),
        out_shape=(jax.ShapeDtypeStruct((E, D), F32),
                   jax.ShapeDtypeStruct((N, D), F32)),
        scratch_shapes=[
            pltpu.VMEM((N, 2 * D), F32),
            pltpu.VMEM((N, 2 * D), F32),
            pltpu.VMEM((E, 1), F32),
            pltpu.VMEM((E, D), F32),
            pltpu.VMEM((1, N), jnp.int32),
            pltpu.VMEM((1, N), jnp.int32),
        ],
    )(sub, obj, rel_feat, sub, obj, phrase_feat, W_rel, W_sub, W_obj, W_phr,
      brel, bsub, bobj, bphr, t)
    return out1, upd


def kernel(phrase_feat, rel_feat, rel_conn_mat, target_id, W_rel, b_rel,
           W_sub, b_sub, W_obj, b_obj, W_phr, b_phr):
    return _run(phrase_feat, rel_feat, rel_conn_mat, target_id, W_rel, b_rel,
                W_sub, b_sub, W_obj, b_obj, W_phr, b_phr)


# speedup vs baseline: 1.2218x; 1.0750x over previous
import jax
import jax.numpy as jnp
from jax.experimental import pallas as pl
from jax.experimental.pallas import tpu as pltpu

N = 256
D = 256
E = 4096
EB = 1024
NBLK = E // EB
EPS = 1e-06
F32 = jnp.float32


def _fused_body(conn_ref, rel_ref, conn_all_ref, phr_ref,
                wrel_ref, wsub_ref, wobj_ref, wphr_ref, brel_ref, bsub_ref,
                bobj_ref, bphr_ref, t_ref, upd_ref, out_ref, tsub_s, tobj_s,
                atte_s, upd_s, erow_s, ecol_s):
    i = pl.program_id(0)

    @pl.when(i == 0)
    def _tables():
        phr = phr_ref[...]
        tsub_s[:, :D] = jnp.dot(phr, wrel_ref[:D], preferred_element_type=F32)
        tsub_s[:, D:] = jnp.dot(phr, wsub_ref[:D], preferred_element_type=F32)
        tobj_s[:, :D] = jnp.dot(phr, wrel_ref[D:2 * D],
                                preferred_element_type=F32)
        tobj_s[:, D:] = jnp.dot(phr, wobj_ref[:D], preferred_element_type=F32)
        t = t_ref[0, 0]
        sub_all = conn_all_ref[0, :]
        obj_all = conn_all_ref[1, :]
        iota_e = jax.lax.broadcasted_iota(jnp.int32, (E, N), 0)
        iota_o = jax.lax.broadcasted_iota(jnp.int32, (E, N), 1)
        subc = sub_all[:, None]
        objc = obj_all[:, None]
        rowval = jnp.where(subc == t, iota_e[:, 0:1], -1)
        colval = jnp.where(objc == t, iota_e[:, 0:1], -1)
        erow_s[0, :] = jnp.max(jnp.where(objc == iota_o, rowval, -1), axis=0)
        ecol_s[0, :] = jnp.max(jnp.where(subc == iota_o, colval, -1), axis=0)

    sub = conn_ref[0, :]
    obj = conn_ref[1, :]
    iota_n = jax.lax.broadcasted_iota(jnp.int32, (EB, N), 1)
    oh_sub = (sub[:, None] == iota_n).astype(F32)
    oh_obj = (obj[:, None] == iota_n).astype(F32)
    gsub = jnp.dot(oh_sub, tsub_s[...], preferred_element_type=F32)
    gobj = jnp.dot(oh_obj, tobj_s[...], preferred_element_type=F32)
    upd = (gsub[:, :D] + gobj[:, :D] + brel_ref[...][None, :]
           + jnp.dot(rel_ref[...], wrel_ref[2 * D:],
                     preferred_element_type=F32))
    upd_ref[...] = upd
    upd_s[pl.ds(i * EB, EB), :] = upd
    ts = gsub[:, D:] + bsub_ref[...][None, :] + jnp.dot(
        upd, wsub_ref[D:], preferred_element_type=F32)
    to = gobj[:, D:] + bobj_ref[...][None, :] + jnp.dot(
        upd, wobj_ref[D:], preferred_element_type=F32)
    ones_col = jnp.ones((D, 1), dtype=F32)
    atte_s[pl.ds(i * EB, EB), :] = jnp.dot(ts * to, ones_col,
                                           preferred_element_type=F32) * (
                                               1.0 / (D ** 0.5))

    @pl.when(i == NBLK - 1)
    def _context():
        t = t_ref[0, 0]
        iota_e = jax.lax.broadcasted_iota(jnp.int32, (E, N), 0)
        e_row = erow_s[0, :]
        e_col = ecol_s[0, :]
        sel_row = (iota_e == e_row[None, :]).astype(F32)
        sel_col = (iota_e == e_col[None, :]).astype(F32)
        atte_col = atte_s[...]
        a_row = jax.lax.dot_general(atte_col, sel_row, (((0,), (0,)), ((), ())),
                                    preferred_element_type=F32)[0]
        a_col = jax.lax.dot_general(atte_col, sel_col, (((0,), (0,)), ((), ())),
                                    preferred_element_type=F32)[0]
        mask_row = (e_row >= 0).astype(F32)
        mask_col = (e_col >= 0).astype(F32)

        def msm(vec, mask):
            mv = vec * mask
            ex = jnp.exp(mv - jnp.max(mv)) * mask
            return ex / (jnp.sum(ex) + EPS)

        w_row = msm(a_row, mask_row)
        w_col = msm(a_col, mask_col)
        weff = (jnp.dot(sel_row, w_row[:, None], preferred_element_type=F32)
                + jnp.dot(sel_col, w_col[:, None], preferred_element_type=F32))
        ctx1 = jnp.dot((w_row + w_col)[None, :], phr_ref[...],
                       preferred_element_type=F32)
        ctx2 = jax.lax.dot_general(weff, upd_s[...], (((0,), (0,)), ((), ())),
                                   preferred_element_type=F32)
        delta = (jnp.dot(ctx1, wphr_ref[:D], preferred_element_type=F32)
                 + jnp.dot(ctx2, wphr_ref[D:], preferred_element_type=F32)
                 + bphr_ref[...][None, :])
        row_is_t = jax.lax.broadcasted_iota(jnp.int32, (N, 1), 0) == t
        out_ref[...] = phr_ref[...] + jnp.where(row_is_t, delta, 0.0)


@jax.jit
def _run(phrase_feat, rel_feat, rel_conn_mat, target_id, W_rel, b_rel, W_sub,
         b_sub, W_obj, b_obj, W_phr, b_phr):
    conn = rel_conn_mat.astype(jnp.int32)
    t = jnp.asarray(target_id, jnp.int32).reshape(1, 1)

    full = lambda shape: pl.BlockSpec(shape, lambda i: tuple(0 for _ in shape))
    upd, out1 = pl.pallas_call(
        _fused_body,
        grid=(NBLK,),
        in_specs=[
            pl.BlockSpec((2, EB), lambda i: (0, i)),
            pl.BlockSpec((EB, D), lambda i: (i, 0)),
            full((2, E)),
            full((N, D)),
            full((3 * D, D)),
            full((2 * D, D)),
            full((2 * D, D)),
            full((2 * D, D)),
            full((D,)),
            full((D,)),
            full((D,)),
            full((D,)),
            full((1, 1)),
        ],
        out_specs=(pl.BlockSpec((EB, D), lambda i: (i, 0)), full((N, D))),
        out_shape=(jax.ShapeDtypeStruct((E, D), F32),
                   jax.ShapeDtypeStruct((N, D), F32)),
        scratch_shapes=[
            pltpu.VMEM((N, 2 * D), F32),
            pltpu.VMEM((N, 2 * D), F32),
            pltpu.VMEM((E, 1), F32),
            pltpu.VMEM((E, D), F32),
            pltpu.VMEM((1, N), jnp.int32),
            pltpu.VMEM((1, N), jnp.int32),
        ],
    )(conn, rel_feat, conn, phrase_feat, W_rel, W_sub, W_obj, W_phr,
      b_rel, b_sub, b_obj, b_phr, t)
    return out1, upd


def kernel(phrase_feat, rel_feat, rel_conn_mat, target_id, W_rel, b_rel,
           W_sub, b_sub, W_obj, b_obj, W_phr, b_phr):
    return _run(phrase_feat, rel_feat, rel_conn_mat, target_id, W_rel, b_rel,
                W_sub, b_sub, W_obj, b_obj, W_phr, b_phr)


# speedup vs baseline: 1.2222x; 1.0004x over previous
import jax
import jax.numpy as jnp
from jax.experimental import pallas as pl
from jax.experimental.pallas import tpu as pltpu

N = 256
D = 256
E = 4096
EB = 1024
NBLK = E // EB
EPS = 1e-06
F32 = jnp.float32
BF16 = jnp.bfloat16


def _fused_body(conn_ref, rel_ref, conn_all_ref, phr_ref,
                wrel_ref, wsub_ref, wobj_ref, wphr_ref, brel_ref, bsub_ref,
                bobj_ref, bphr_ref, t_ref, upd_ref, out_ref, tsub_s, tobj_s,
                atte_s, upd_s, erow_s, ecol_s):
    i = pl.program_id(0)

    @pl.when(i == 0)
    def _tables():
        phr = phr_ref[...]
        tsub_s[:, :D] = jnp.dot(phr, wrel_ref[:D],
                                preferred_element_type=F32).astype(BF16)
        tsub_s[:, D:] = jnp.dot(phr, wsub_ref[:D],
                                preferred_element_type=F32).astype(BF16)
        tobj_s[:, :D] = jnp.dot(phr, wrel_ref[D:2 * D],
                                preferred_element_type=F32).astype(BF16)
        tobj_s[:, D:] = jnp.dot(phr, wobj_ref[:D],
                                preferred_element_type=F32).astype(BF16)
        t = t_ref[0, 0]
        sub_all = conn_all_ref[0, :]
        obj_all = conn_all_ref[1, :]
        iota_e = jax.lax.broadcasted_iota(jnp.int32, (E, N), 0)
        iota_o = jax.lax.broadcasted_iota(jnp.int32, (E, N), 1)
        subc = sub_all[:, None]
        objc = obj_all[:, None]
        rowval = jnp.where(subc == t, iota_e[:, 0:1], -1)
        colval = jnp.where(objc == t, iota_e[:, 0:1], -1)
        erow_s[0, :] = jnp.max(jnp.where(objc == iota_o, rowval, -1), axis=0)
        ecol_s[0, :] = jnp.max(jnp.where(subc == iota_o, colval, -1), axis=0)

    sub = conn_ref[0, :]
    obj = conn_ref[1, :]
    iota_n = jax.lax.broadcasted_iota(jnp.int32, (EB, N), 1)
    oh_sub = (sub[:, None] == iota_n).astype(BF16)
    oh_obj = (obj[:, None] == iota_n).astype(BF16)
    gsub = jnp.dot(oh_sub, tsub_s[...], preferred_element_type=F32)
    gobj = jnp.dot(oh_obj, tobj_s[...], preferred_element_type=F32)
    upd = (gsub[:, :D] + gobj[:, :D] + brel_ref[...][None, :]
           + jnp.dot(rel_ref[...].astype(BF16),
                     wrel_ref[2 * D:].astype(BF16),
                     preferred_element_type=F32))
    upd_ref[...] = upd
    upd_s[pl.ds(i * EB, EB), :] = upd
    ts = gsub[:, D:] + bsub_ref[...][None, :] + jnp.dot(
        upd.astype(BF16), wsub_ref[D:].astype(BF16),
        preferred_element_type=F32)
    to = gobj[:, D:] + bobj_ref[...][None, :] + jnp.dot(
        upd.astype(BF16), wobj_ref[D:].astype(BF16),
        preferred_element_type=F32)
    ones_col = jnp.ones((D, 1), dtype=F32)
    atte_s[pl.ds(i * EB, EB), :] = jnp.dot(ts * to, ones_col,
                                           preferred_element_type=F32) * (
                                               1.0 / (D ** 0.5))

    @pl.when(i == NBLK - 1)
    def _context():
        t = t_ref[0, 0]
        iota_e = jax.lax.broadcasted_iota(jnp.int32, (E, N), 0)
        e_row = erow_s[0, :]
        e_col = ecol_s[0, :]
        sel_row = (iota_e == e_row[None, :]).astype(F32)
        sel_col = (iota_e == e_col[None, :]).astype(F32)
        atte_col = atte_s[...]
        a_row = jax.lax.dot_general(atte_col, sel_row, (((0,), (0,)), ((), ())),
                                    preferred_element_type=F32)[0]
        a_col = jax.lax.dot_general(atte_col, sel_col, (((0,), (0,)), ((), ())),
                                    preferred_element_type=F32)[0]
        mask_row = (e_row >= 0).astype(F32)
        mask_col = (e_col >= 0).astype(F32)

        def msm(vec, mask):
            mv = vec * mask
            ex = jnp.exp(mv - jnp.max(mv)) * mask
            return ex / (jnp.sum(ex) + EPS)

        w_row = msm(a_row, mask_row)
        w_col = msm(a_col, mask_col)
        weff = (jnp.dot(sel_row, w_row[:, None], preferred_element_type=F32)
                + jnp.dot(sel_col, w_col[:, None], preferred_element_type=F32))
        ctx1 = jnp.dot((w_row + w_col)[None, :], phr_ref[...],
                       preferred_element_type=F32)
        ctx2 = jax.lax.dot_general(weff, upd_s[...], (((0,), (0,)), ((), ())),
                                   preferred_element_type=F32)
        delta = (jnp.dot(ctx1, wphr_ref[:D], preferred_element_type=F32)
                 + jnp.dot(ctx2, wphr_ref[D:], preferred_element_type=F32)
                 + bphr_ref[...][None, :])
        row_is_t = jax.lax.broadcasted_iota(jnp.int32, (N, 1), 0) == t
        out_ref[...] = phr_ref[...] + jnp.where(row_is_t, delta, 0.0)


@jax.jit
def _run(phrase_feat, rel_feat, rel_conn_mat, target_id, W_rel, b_rel, W_sub,
         b_sub, W_obj, b_obj, W_phr, b_phr):
    conn = rel_conn_mat.astype(jnp.int32)
    t = jnp.asarray(target_id, jnp.int32).reshape(1, 1)

    full = lambda shape: pl.BlockSpec(shape, lambda i: tuple(0 for _ in shape))
    upd, out1 = pl.pallas_call(
        _fused_body,
        grid=(NBLK,),
        in_specs=[
            pl.BlockSpec((2, EB), lambda i: (0, i)),
            pl.BlockSpec((EB, D), lambda i: (i, 0)),
            full((2, E)),
            full((N, D)),
            full((3 * D, D)),
            full((2 * D, D)),
            full((2 * D, D)),
            full((2 * D, D)),
            full((D,)),
            full((D,)),
            full((D,)),
            full((D,)),
            full((1, 1)),
        ],
        out_specs=(pl.BlockSpec((EB, D), lambda i: (i, 0)), full((N, D))),
        out_shape=(jax.ShapeDtypeStruct((E, D), F32),
                   jax.ShapeDtypeStruct((N, D), F32)),
        scratch_shapes=[
            pltpu.VMEM((N, 2 * D), BF16),
            pltpu.VMEM((N, 2 * D), BF16),
            pltpu.VMEM((E, 1), F32),
            pltpu.VMEM((E, D), F32),
            pltpu.VMEM((1, N), jnp.int32),
            pltpu.VMEM((1, N), jnp.int32),
        ],
    )(conn, rel_feat, conn, phrase_feat, W_rel, W_sub, W_obj, W_phr,
      b_rel, b_sub, b_obj, b_phr, t)
    return out1, upd


def kernel(phrase_feat, rel_feat, rel_conn_mat, target_id, W_rel, b_rel,
           W_sub, b_sub, W_obj, b_obj, W_phr, b_phr):
    return _run(phrase_feat, rel_feat, rel_conn_mat, target_id, W_rel, b_rel,
                W_sub, b_sub, W_obj, b_obj, W_phr, b_phr)


# speedup vs baseline: 1.2794x; 1.0468x over previous
import jax
import jax.numpy as jnp
from jax.experimental import pallas as pl
from jax.experimental.pallas import tpu as pltpu

N = 256
D = 256
E = 4096
EB = 2048
NBLK = E // EB
EPS = 1e-06
F32 = jnp.float32


def _fused_body(conn_ref, rel_ref, conn_all_ref, phr_ref,
                wrel_ref, wsub_ref, wobj_ref, wphr_ref, brel_ref, bsub_ref,
                bobj_ref, bphr_ref, t_ref, upd_ref, out_ref, tsub_s, tobj_s,
                atte_s, upd_s, erow_s, ecol_s):
    i = pl.program_id(0)

    @pl.when(i == 0)
    def _tables():
        phr = phr_ref[...]
        tsub_s[:, :D] = jnp.dot(phr, wrel_ref[:D], preferred_element_type=F32)
        tsub_s[:, D:] = jnp.dot(phr, wsub_ref[:D], preferred_element_type=F32)
        tobj_s[:, :D] = jnp.dot(phr, wrel_ref[D:2 * D],
                                preferred_element_type=F32)
        tobj_s[:, D:] = jnp.dot(phr, wobj_ref[:D], preferred_element_type=F32)
        t = t_ref[0, 0]
        sub_all = conn_all_ref[0, :]
        obj_all = conn_all_ref[1, :]
        iota_e = jax.lax.broadcasted_iota(jnp.int32, (E, N), 0)
        iota_o = jax.lax.broadcasted_iota(jnp.int32, (E, N), 1)
        subc = sub_all[:, None]
        objc = obj_all[:, None]
        rowval = jnp.where(subc == t, iota_e[:, 0:1], -1)
        colval = jnp.where(objc == t, iota_e[:, 0:1], -1)
        erow_s[0, :] = jnp.max(jnp.where(objc == iota_o, rowval, -1), axis=0)
        ecol_s[0, :] = jnp.max(jnp.where(subc == iota_o, colval, -1), axis=0)

    sub = conn_ref[0, :]
    obj = conn_ref[1, :]
    iota_n = jax.lax.broadcasted_iota(jnp.int32, (EB, N), 1)
    oh_sub = (sub[:, None] == iota_n).astype(F32)
    oh_obj = (obj[:, None] == iota_n).astype(F32)
    gsub = jnp.dot(oh_sub, tsub_s[...], preferred_element_type=F32)
    gobj = jnp.dot(oh_obj, tobj_s[...], preferred_element_type=F32)
    upd = (gsub[:, :D] + gobj[:, :D] + brel_ref[...][None, :]
           + jnp.dot(rel_ref[...], wrel_ref[2 * D:],
                     preferred_element_type=F32))
    upd_ref[...] = upd
    upd_s[pl.ds(i * EB, EB), :] = upd
    ts = gsub[:, D:] + bsub_ref[...][None, :] + jnp.dot(
        upd, wsub_ref[D:], preferred_element_type=F32)
    to = gobj[:, D:] + bobj_ref[...][None, :] + jnp.dot(
        upd, wobj_ref[D:], preferred_element_type=F32)
    ones_col = jnp.ones((D, 1), dtype=F32)
    atte_s[pl.ds(i * EB, EB), :] = jnp.dot(ts * to, ones_col,
                                           preferred_element_type=F32) * (
                                               1.0 / (D ** 0.5))

    @pl.when(i == NBLK - 1)
    def _context():
        t = t_ref[0, 0]
        iota_e = jax.lax.broadcasted_iota(jnp.int32, (E, N), 0)
        e_row = erow_s[0, :]
        e_col = ecol_s[0, :]
        sel_row = (iota_e == e_row[None, :]).astype(F32)
        sel_col = (iota_e == e_col[None, :]).astype(F32)
        atte_col = atte_s[...]
        a_row = jax.lax.dot_general(atte_col, sel_row, (((0,), (0,)), ((), ())),
                                    preferred_element_type=F32)[0]
        a_col = jax.lax.dot_general(atte_col, sel_col, (((0,), (0,)), ((), ())),
                                    preferred_element_type=F32)[0]
        mask_row = (e_row >= 0).astype(F32)
        mask_col = (e_col >= 0).astype(F32)

        def msm(vec, mask):
            mv = vec * mask
            ex = jnp.exp(mv - jnp.max(mv)) * mask
            return ex / (jnp.sum(ex) + EPS)

        w_row = msm(a_row, mask_row)
        w_col = msm(a_col, mask_col)
        weff = (jnp.dot(sel_row, w_row[:, None], preferred_element_type=F32)
                + jnp.dot(sel_col, w_col[:, None], preferred_element_type=F32))
        ctx1 = jnp.dot((w_row + w_col)[None, :], phr_ref[...],
                       preferred_element_type=F32)
        ctx2 = jax.lax.dot_general(weff, upd_s[...], (((0,), (0,)), ((), ())),
                                   preferred_element_type=F32)
        delta = (jnp.dot(ctx1, wphr_ref[:D], preferred_element_type=F32)
                 + jnp.dot(ctx2, wphr_ref[D:], preferred_element_type=F32)
                 + bphr_ref[...][None, :])
        row_is_t = jax.lax.broadcasted_iota(jnp.int32, (N, 1), 0) == t
        out_ref[...] = phr_ref[...] + jnp.where(row_is_t, delta, 0.0)


@jax.jit
def _run(phrase_feat, rel_feat, rel_conn_mat, target_id, W_rel, b_rel, W_sub,
         b_sub, W_obj, b_obj, W_phr, b_phr):
    conn = rel_conn_mat.astype(jnp.int32)
    t = jnp.asarray(target_id, jnp.int32).reshape(1, 1)

    full = lambda shape: pl.BlockSpec(shape, lambda i: tuple(0 for _ in shape))
    upd, out1 = pl.pallas_call(
        _fused_body,
        grid=(NBLK,),
        in_specs=[
            pl.BlockSpec((2, EB), lambda i: (0, i)),
            pl.BlockSpec((EB, D), lambda i: (i, 0)),
            full((2, E)),
            full((N, D)),
            full((3 * D, D)),
            full((2 * D, D)),
            full((2 * D, D)),
            full((2 * D, D)),
            full((D,)),
            full((D,)),
            full((D,)),
            full((D,)),
            full((1, 1)),
        ],
        out_specs=(pl.BlockSpec((EB, D), lambda i: (i, 0)), full((N, D))),
        out_shape=(jax.ShapeDtypeStruct((E, D), F32),
                   jax.ShapeDtypeStruct((N, D), F32)),
        scratch_shapes=[
            pltpu.VMEM((N, 2 * D), F32),
            pltpu.VMEM((N, 2 * D), F32),
            pltpu.VMEM((E, 1), F32),
            pltpu.VMEM((E, D), F32),
            pltpu.VMEM((1, N), jnp.int32),
            pltpu.VMEM((1, N), jnp.int32),
        ],
    )(conn, rel_feat, conn, phrase_feat, W_rel, W_sub, W_obj, W_phr,
      b_rel, b_sub, b_obj, b_phr, t)
    return out1, upd


def kernel(phrase_feat, rel_feat, rel_conn_mat, target_id, W_rel, b_rel,
           W_sub, b_sub, W_obj, b_obj, W_phr, b_phr):
    return _run(phrase_feat, rel_feat, rel_conn_mat, target_id, W_rel, b_rel,
                W_sub, b_sub, W_obj, b_obj, W_phr, b_phr)
